# baseline (speedup 1.0000x reference)
import functools

import jax
import jax.numpy as jnp
from jax import lax
from jax.experimental import pallas as pl
from jax.experimental.pallas import tpu as pltpu
from jax.experimental.pallas import tpu_sc as plsc

N_NODES = 10000
D = 128
E2 = 640000
NUM_INPUT = 4096

NC = 2
NS = 16
NW = NC * NS
NB = 4 * NW
NPP = 80
NPAD = NB * NPP
NTC = 10240
CHUNK = 128
DUMP = NPP

_FMAX = 3.4e38


def _sc_aggregate(x, src_s, dstl_s, off, with_deg):
    mesh = plsc.VectorSubcoreMesh(core_axis_name="c", subcore_axis_name="s")
    out_t = jax.ShapeDtypeStruct((NTC, D), jnp.float32)
    outs = (out_t, out_t, out_t, out_t)
    if with_deg:
        outs = outs + (jax.ShapeDtypeStruct((NTC, 16), jnp.float32),)

    @functools.partial(
        pl.kernel,
        mesh=mesh,
        out_type=outs,
        scratch_types=[
            pltpu.VMEM((NPP + 1, D), jnp.float32),
            pltpu.VMEM((NPP + 1, D), jnp.float32),
            pltpu.VMEM((NPP + 1, D), jnp.float32),
            pltpu.VMEM((NPP + 1, D), jnp.float32),
            pltpu.VMEM((NPP + 1, 16), jnp.float32),
            pltpu.VMEM((CHUNK + 16, D), jnp.float32),
            pltpu.VMEM((CHUNK + 16, D), jnp.float32),
            pltpu.VMEM((CHUNK,), jnp.int32),
            pltpu.VMEM((CHUNK,), jnp.int32),
            pltpu.VMEM((CHUNK + 16,), jnp.int32),
            pltpu.VMEM((CHUNK + 16,), jnp.int32),
            pltpu.VMEM((160,), jnp.int32),
            pltpu.SemaphoreType.DMA,
            pltpu.SemaphoreType.DMA,
            pltpu.SemaphoreType.DMA,
            pltpu.SemaphoreType.DMA,
        ],
    )
    def agg(x_hbm, srcs_hbm, dstl_hbm, off_hbm, *refs):
        if with_deg:
            (oS, oQ, oM, oN, oD, accS, accQ, accM, accN, accD,
             rows0, rows1, sidx0, sidx1, dloc0, dloc1, offv,
             ssem0, ssem1, gsem0, gsem1) = refs
        else:
            (oS, oQ, oM, oN, accS, accQ, accM, accN, accD,
             rows0, rows1, sidx0, sidx1, dloc0, dloc1, offv,
             ssem0, ssem1, gsem0, gsem1) = refs
        bufs = ((sidx0, dloc0, rows0, ssem0, gsem0),
                (sidx1, dloc1, rows1, ssem1, gsem1))
        wid = lax.axis_index("s") * NC + lax.axis_index("c")
        pltpu.sync_copy(off_hbm, offv)

        zeros16 = jnp.zeros((16,), jnp.float32)
        ones16 = jnp.ones((16,), jnp.float32)
        big = jnp.full((16,), _FMAX, jnp.float32)

        def do_bucket(bi, _):
            b = wid * 4 + bi
            start = offv[pl.ds(b, 16)][0]
            end = offv[pl.ds(b + 1, 16)][0]
            abase = lax.div(start, CHUNK) * CHUNK
            nchunks = lax.div(end - abase + (CHUNK - 1), CHUNK)

            def zrow(r, _):
                def zcol(j, _):
                    sl = pl.ds(j * 16, 16)
                    accS[r, sl] = zeros16
                    accQ[r, sl] = zeros16
                    accM[r, sl] = -big
                    accN[r, sl] = big
                    return 0
                lax.fori_loop(0, D // 16, zcol, 0, unroll=8)
                if with_deg:
                    accD[r, pl.ds(0, 16)] = zeros16
                return 0
            lax.fori_loop(0, NPP + 1, zrow, 0)

            NJ = D // 16
            neg1 = jnp.full((16,), -1, jnp.int32)

            def cbase_of(k):
                return pl.multiple_of(abase + k * CHUNK, CHUNK)

            def idx_copies(k, B):
                sidx, dloc = B[0], B[1]
                cb = cbase_of(k)
                return (
                    pltpu.make_async_copy(
                        srcs_hbm.at[pl.ds(cb, CHUNK)], sidx, B[3]),
                    pltpu.make_async_copy(
                        dstl_hbm.at[pl.ds(cb, CHUNK)],
                        dloc.at[pl.ds(0, CHUNK)], B[3]))

            def issue_idx(k, B):
                @pl.when(k < nchunks)
                def _():
                    for cp in idx_copies(k, B):
                        cp.start()

            def fix_and_gather(k, B):
                @pl.when(k < nchunks)
                def _():
                    sidx, dloc = B[0], B[1]
                    for cp in idx_copies(k, B):
                        cp.wait()
                    cb = cbase_of(k)
                    lo = start - cb
                    hi = end - cb
                    hi_c = jnp.minimum(hi, CHUNK)
                    dloc[pl.ds(CHUNK, 16)] = neg1
                    def fix(j, _):
                        sl = pl.ds(j * 16, 16)
                        lane = lax.iota(jnp.int32, 16) + j * 16
                        valid = (lane >= lo) & (lane < hi)
                        sidx[sl] = jnp.where(valid, sidx[sl], 0)
                        dloc[sl] = jnp.where(lane < hi_c, dloc[sl], -1)
                        return 0
                    lax.fori_loop(0, CHUNK // 16, fix, 0, unroll=8)
                    pltpu.make_async_copy(
                        x_hbm.at[sidx], B[2].at[pl.ds(0, CHUNK)],
                        B[4]).start()

            def compute_chunk(k, B):
                sidx, dloc, rows = B[0], B[1], B[2]
                cb = cbase_of(k)
                lo = start - cb
                hi = end - cb
                lo_c = jnp.maximum(lo, 0)
                hi_c = jnp.minimum(hi, CHUNK)

                ngroups = lax.div(hi_c - lo_c + 15, 16)

                def do_group(g, _):
                    e0 = lo_c + g * 16
                    gv = dloc[pl.ds(e0, 16)]
                    d0 = gv[0]
                    d15 = gv[15]

                    @pl.when(d0 == d15)
                    def fast():
                        rS = tuple(accS[d0, pl.ds(j * 16, 16)]
                                   for j in range(NJ))
                        rQ = tuple(accQ[d0, pl.ds(j * 16, 16)]
                                   for j in range(NJ))
                        rM = tuple(accM[d0, pl.ds(j * 16, 16)]
                                   for j in range(NJ))
                        rN = tuple(accN[d0, pl.ds(j * 16, 16)]
                                   for j in range(NJ))

                        def inner(i, regs):
                            e = e0 + i
                            rS, rQ, rM, rN = regs
                            r = tuple(rows[e, pl.ds(j * 16, 16)]
                                      for j in range(NJ))
                            return (
                                tuple(rS[j] + r[j] for j in range(NJ)),
                                tuple(rQ[j] + r[j] * r[j]
                                      for j in range(NJ)),
                                tuple(jnp.maximum(rM[j], r[j])
                                      for j in range(NJ)),
                                tuple(jnp.minimum(rN[j], r[j])
                                      for j in range(NJ)))

                        rS, rQ, rM, rN = lax.fori_loop(
                            0, 16, inner, (rS, rQ, rM, rN), unroll=8)
                        for j in range(NJ):
                            sl = pl.ds(j * 16, 16)
                            accS[d0, sl] = rS[j]
                            accQ[d0, sl] = rQ[j]
                            accM[d0, sl] = rM[j]
                            accN[d0, sl] = rN[j]
                        if with_deg:
                            accD[d0, pl.ds(0, 16)] += ones16 * 16.0

                    @pl.when(d0 != d15)
                    def slow():
                        def pe(i, _):
                            e = e0 + i
                            dr = dloc[pl.ds(e, 16)][0]
                            d = jnp.where((e < hi_c) & (dr >= 0), dr, DUMP)

                            def upd(j, _):
                                sl = pl.ds(j * 16, 16)
                                r = rows[e, sl]
                                accS[d, sl] += r
                                accQ[d, sl] += r * r
                                accM[d, sl] = jnp.maximum(accM[d, sl], r)
                                accN[d, sl] = jnp.minimum(accN[d, sl], r)
                                return 0
                            lax.fori_loop(0, NJ, upd, 0, unroll=8)
                            if with_deg:
                                accD[d, pl.ds(0, 16)] += ones16
                            return 0
                        lax.fori_loop(0, 16, pe, 0, unroll=4)
                    return 0

                lax.fori_loop(0, ngroups, do_group, 0)

            issue_idx(0, bufs[0])
            fix_and_gather(0, bufs[0])
            issue_idx(1, bufs[1])

            def pipe(k2, _):
                for ph in range(2):
                    k = k2 * 2 + ph
                    B = bufs[ph]
                    Bn = bufs[1 - ph]

                    @pl.when(k < nchunks)
                    def _():
                        pltpu.make_async_copy(
                            x_hbm.at[B[0]], B[2].at[pl.ds(0, CHUNK)],
                            B[4]).wait()
                        fix_and_gather(k + 1, Bn)
                        compute_chunk(k, B)
                        issue_idx(k + 2, B)
                return 0

            lax.fori_loop(0, lax.div(nchunks + 1, 2), pipe, 0)

            nb = b * NPP
            pltpu.sync_copy(accS.at[pl.ds(0, NPP)], oS.at[pl.ds(nb, NPP)])
            pltpu.sync_copy(accQ.at[pl.ds(0, NPP)], oQ.at[pl.ds(nb, NPP)])
            pltpu.sync_copy(accM.at[pl.ds(0, NPP)], oM.at[pl.ds(nb, NPP)])
            pltpu.sync_copy(accN.at[pl.ds(0, NPP)], oN.at[pl.ds(nb, NPP)])
            if with_deg:
                pltpu.sync_copy(accD.at[pl.ds(0, NPP)], oD.at[pl.ds(nb, NPP)])
            return 0

        lax.fori_loop(0, 4, do_bucket, 0)

    return agg(x, src_s, dstl_s, off)


def _tc_scalars(deg):
    def body(deg_ref, ic_ref, s1_ref, s2_ref, hb_ref):
        dg = deg_ref[...]
        node = lax.broadcasted_iota(jnp.int32, (1, NTC), 1)
        real = node < N_NODES
        lg = jnp.where(real, jnp.log(jnp.where(real, dg, 0.0) + 1.0), 0.0)
        mean_lg = jnp.sum(lg) / N_NODES
        scale = lg / jnp.clip(mean_lg, 1e-6, None)
        s2 = 1.0 / jnp.clip(scale, 1e-2, None)
        ic = 1.0 / jnp.clip(dg, 1.0, None)
        hb = jnp.where(dg > 0, 1.0, 0.0)
        ic_ref[...] = ic.reshape(NTC, 1)
        s1_ref[...] = scale.reshape(NTC, 1)
        s2_ref[...] = s2.reshape(NTC, 1)
        hb_ref[...] = hb.reshape(NTC, 1)

    o = jax.ShapeDtypeStruct((NTC, 1), jnp.float32)
    return pl.pallas_call(body, out_shape=(o, o, o, o))(deg)


def _tc_dense(x, S, Q, MX, MN, ic, s1, s2, hb, Wself, Wf0, Wf1, Wf2, b):
    BLK = 512

    def body(x_ref, S_ref, Q_ref, MX_ref, MN_ref, ic_ref, s1_ref, s2_ref,
             hb_ref, Wself_ref, Wf0_ref, Wf1_ref, Wf2_ref, b_ref, o_ref):
        xv = x_ref[...]
        ic_ = ic_ref[...]
        hb_ = hb_ref[...]
        mean = S_ref[...] * ic_
        sqm = Q_ref[...] * ic_
        std = jnp.sqrt(jnp.clip(sqm - mean * mean, 0.0, None))
        mx = jnp.where(hb_ > 0, MX_ref[...], 0.0)
        mn = jnp.where(hb_ > 0, MN_ref[...], 0.0)
        feat = jnp.concatenate([mean, mx, mn, std], axis=1)
        acc = jnp.dot(xv, Wself_ref[...], preferred_element_type=jnp.float32)
        acc += jnp.dot(feat, Wf0_ref[...], preferred_element_type=jnp.float32)
        acc += jnp.dot(feat * s1_ref[...], Wf1_ref[...],
                       preferred_element_type=jnp.float32)
        acc += jnp.dot(feat * s2_ref[...], Wf2_ref[...],
                       preferred_element_type=jnp.float32)
        o_ref[...] = jax.nn.relu(acc + b_ref[...]) + xv

    n_blk = NTC // BLK
    row = lambda i: (i, 0)
    fix = lambda i: (0, 0)
    rspec = pl.BlockSpec((BLK, D), row)
    sspec = pl.BlockSpec((BLK, 1), row)
    return pl.pallas_call(
        body,
        grid=(n_blk,),
        in_specs=[rspec, rspec, rspec, rspec, rspec, sspec, sspec, sspec,
                  sspec,
                  pl.BlockSpec((D, D), fix),
                  pl.BlockSpec((4 * D, D), fix),
                  pl.BlockSpec((4 * D, D), fix),
                  pl.BlockSpec((4 * D, D), fix),
                  pl.BlockSpec((1, D), fix)],
        out_specs=rspec,
        out_shape=jax.ShapeDtypeStruct((NTC, D), jnp.float32),
    )(x, S, Q, MX, MN, ic, s1, s2, hb, Wself, Wf0, Wf1, Wf2, b)


def kernel(input_embeds, edge_index, input_index, W0, b0, W1, b1, W2, b2):
    src = jnp.concatenate([edge_index[0], edge_index[1]])
    dst = jnp.concatenate([edge_index[1], edge_index[0]])

    perm = jnp.argsort(dst)
    src_s = src[perm]
    dst_s = dst[perm]
    dstl_s = dst_s % NPP
    bounds = jnp.arange(0, NB + 1, dtype=jnp.int32) * NPP
    off = jnp.searchsorted(dst_s, bounds, side="left").astype(jnp.int32)
    off = jnp.concatenate([off, jnp.full((160 - NB - 1,), E2, jnp.int32)])

    ii = jnp.arange(NUM_INPUT, dtype=jnp.int32)
    winner = jax.ops.segment_max(ii, input_index, num_segments=N_NODES)
    winner = jnp.where(winner >= 0, winner, NUM_INPUT)
    emb_pad = jnp.concatenate(
        [input_embeds, jnp.zeros((1, D), jnp.float32)], axis=0)
    x0 = emb_pad[winner]
    x0 = jnp.concatenate([x0, jnp.zeros((NTC - N_NODES, D), jnp.float32)])

    layers = []
    for W, b in ((W0, b0), (W1, b1), (W2, b2)):
        Wself = W[:D]
        Wf = W[D:].reshape(4 * D, 3, D)
        layers.append((Wself, Wf[:, 0, :], Wf[:, 1, :], Wf[:, 2, :],
                       b.reshape(1, D)))

    x = x0
    ic = s1 = s2 = hb = None
    for li, (Wself, Wf0, Wf1, Wf2, bb) in enumerate(layers):
        if li == 0:
            S, Q, MX, MN, Dg = _sc_aggregate(x, src_s, dstl_s, off, True)
            deg_p = Dg[:, 0].reshape(1, NTC)
            ic, s1, s2, hb = _tc_scalars(deg_p)
        else:
            S, Q, MX, MN = _sc_aggregate(x, src_s, dstl_s, off, False)
        x = _tc_dense(x, S, Q, MX, MN, ic, s1, s2, hb,
                      Wself, Wf0, Wf1, Wf2, bb)

    return x[:N_NODES]


# speedup vs baseline: 1.1666x; 1.1666x over previous
import functools

import jax
import jax.numpy as jnp
from jax import lax
from jax.experimental import pallas as pl
from jax.experimental.pallas import tpu as pltpu
from jax.experimental.pallas import tpu_sc as plsc

N_NODES = 10000
D = 128
E2 = 640000
NUM_INPUT = 4096

NC = 2
NS = 16
NW = NC * NS
NB = 4 * NW
NPP = 80
NPAD = NB * NPP
NTC = 10240
CHUNK = 128
DUMP = NPP

_FMAX = 3.4e38


def _sc_aggregate(x, src_s, dstl_s, off, with_deg):
    mesh = plsc.VectorSubcoreMesh(core_axis_name="c", subcore_axis_name="s")
    out_t = jax.ShapeDtypeStruct((NTC, D), jnp.float32)
    outs = (out_t, out_t, out_t, out_t)
    if with_deg:
        outs = outs + (jax.ShapeDtypeStruct((NTC, 16), jnp.float32),)

    @functools.partial(
        pl.kernel,
        mesh=mesh,
        out_type=outs,
        scratch_types=[
            pltpu.VMEM((NPP + 1, D), jnp.float32),
            pltpu.VMEM((NPP + 1, D), jnp.float32),
            pltpu.VMEM((NPP + 1, D), jnp.float32),
            pltpu.VMEM((NPP + 1, D), jnp.float32),
            pltpu.VMEM((NPP + 1, 16), jnp.float32),
            pltpu.VMEM((CHUNK + 16, D), jnp.float32),
            pltpu.VMEM((CHUNK + 16, D), jnp.float32),
            pltpu.VMEM((CHUNK,), jnp.int32),
            pltpu.VMEM((CHUNK,), jnp.int32),
            pltpu.VMEM((CHUNK + 16,), jnp.int32),
            pltpu.VMEM((CHUNK + 16,), jnp.int32),
            pltpu.VMEM((160,), jnp.int32),
            pltpu.SemaphoreType.DMA,
            pltpu.SemaphoreType.DMA,
            pltpu.SemaphoreType.DMA,
            pltpu.SemaphoreType.DMA,
        ],
    )
    def agg(x_hbm, srcs_hbm, dstl_hbm, off_hbm, *refs):
        if with_deg:
            (oS, oQ, oM, oN, oD, accS, accQ, accM, accN, accD,
             rows0, rows1, sidx0, sidx1, dloc0, dloc1, offv,
             ssem0, ssem1, gsem0, gsem1) = refs
        else:
            (oS, oQ, oM, oN, accS, accQ, accM, accN, accD,
             rows0, rows1, sidx0, sidx1, dloc0, dloc1, offv,
             ssem0, ssem1, gsem0, gsem1) = refs
        bufs = ((sidx0, dloc0, rows0, ssem0, gsem0),
                (sidx1, dloc1, rows1, ssem1, gsem1))
        wid = lax.axis_index("s") * NC + lax.axis_index("c")
        pltpu.sync_copy(off_hbm, offv)

        zeros16 = jnp.zeros((16,), jnp.float32)
        ones16 = jnp.ones((16,), jnp.float32)
        big = jnp.full((16,), _FMAX, jnp.float32)

        def do_bucket(bi, _):
            b = wid * 4 + bi
            start = offv[pl.ds(b, 16)][0]
            end = offv[pl.ds(b + 1, 16)][0]
            abase = lax.div(start, CHUNK) * CHUNK
            nchunks = lax.div(end - abase + (CHUNK - 1), CHUNK)

            def zrow(r, _):
                def zcol(j, _):
                    sl = pl.ds(j * 16, 16)
                    accS[r, sl] = zeros16
                    accQ[r, sl] = zeros16
                    accM[r, sl] = -big
                    accN[r, sl] = big
                    return 0
                lax.fori_loop(0, D // 16, zcol, 0, unroll=8)
                if with_deg:
                    accD[r, pl.ds(0, 16)] = zeros16
                return 0
            lax.fori_loop(0, NPP + 1, zrow, 0)

            NJ = D // 16
            neg1 = jnp.full((16,), -1, jnp.int32)

            def cbase_of(k):
                return pl.multiple_of(abase + k * CHUNK, CHUNK)

            def idx_copies(k, B):
                sidx, dloc = B[0], B[1]
                cb = cbase_of(k)
                return (
                    pltpu.make_async_copy(
                        srcs_hbm.at[pl.ds(cb, CHUNK)], sidx, B[3]),
                    pltpu.make_async_copy(
                        dstl_hbm.at[pl.ds(cb, CHUNK)],
                        dloc.at[pl.ds(0, CHUNK)], B[3]))

            def issue_idx(k, B):
                @pl.when(k < nchunks)
                def _():
                    for cp in idx_copies(k, B):
                        cp.start()

            def fix_and_gather(k, B):
                @pl.when(k < nchunks)
                def _():
                    sidx, dloc = B[0], B[1]
                    for cp in idx_copies(k, B):
                        cp.wait()
                    cb = cbase_of(k)
                    lo = start - cb
                    hi = end - cb
                    hi_c = jnp.minimum(hi, CHUNK)
                    dloc[pl.ds(CHUNK, 16)] = neg1
                    def fix(j, _):
                        sl = pl.ds(j * 16, 16)
                        lane = lax.iota(jnp.int32, 16) + j * 16
                        valid = (lane >= lo) & (lane < hi)
                        sidx[sl] = jnp.where(valid, sidx[sl], 0)
                        dloc[sl] = jnp.where(lane < hi_c, dloc[sl], -1)
                        return 0
                    lax.fori_loop(0, CHUNK // 16, fix, 0, unroll=8)
                    pltpu.make_async_copy(
                        x_hbm.at[sidx], B[2].at[pl.ds(0, CHUNK)],
                        B[4]).start()

            def compute_chunk(k, B):
                sidx, dloc, rows = B[0], B[1], B[2]
                cb = cbase_of(k)
                lo = start - cb
                hi = end - cb
                lo_c = jnp.maximum(lo, 0)
                hi_c = jnp.minimum(hi, CHUNK)

                ngroups = lax.div(hi_c - lo_c + 15, 16)

                def do_group(g, _):
                    e0 = lo_c + g * 16
                    gv = dloc[pl.ds(e0, 16)]
                    d0 = gv[0]
                    d15 = gv[15]

                    @pl.when(d0 == d15)
                    def fast():
                        rS = tuple(accS[d0, pl.ds(j * 16, 16)]
                                   for j in range(NJ))
                        rQ = tuple(accQ[d0, pl.ds(j * 16, 16)]
                                   for j in range(NJ))
                        rM = tuple(accM[d0, pl.ds(j * 16, 16)]
                                   for j in range(NJ))
                        rN = tuple(accN[d0, pl.ds(j * 16, 16)]
                                   for j in range(NJ))

                        def inner(i, regs):
                            e = e0 + i
                            rS, rQ, rM, rN = regs
                            r = tuple(rows[e, pl.ds(j * 16, 16)]
                                      for j in range(NJ))
                            return (
                                tuple(rS[j] + r[j] for j in range(NJ)),
                                tuple(rQ[j] + r[j] * r[j]
                                      for j in range(NJ)),
                                tuple(jnp.maximum(rM[j], r[j])
                                      for j in range(NJ)),
                                tuple(jnp.minimum(rN[j], r[j])
                                      for j in range(NJ)))

                        rS, rQ, rM, rN = lax.fori_loop(
                            0, 16, inner, (rS, rQ, rM, rN))
                        for j in range(NJ):
                            sl = pl.ds(j * 16, 16)
                            accS[d0, sl] = rS[j]
                            accQ[d0, sl] = rQ[j]
                            accM[d0, sl] = rM[j]
                            accN[d0, sl] = rN[j]
                        if with_deg:
                            accD[d0, pl.ds(0, 16)] += ones16 * 16.0

                    @pl.when(d0 != d15)
                    def slow():
                        def pe(i, _):
                            e = e0 + i
                            dr = dloc[pl.ds(e, 16)][0]
                            d = jnp.where((e < hi_c) & (dr >= 0), dr, DUMP)

                            def upd(j, _):
                                sl = pl.ds(j * 16, 16)
                                r = rows[e, sl]
                                accS[d, sl] += r
                                accQ[d, sl] += r * r
                                accM[d, sl] = jnp.maximum(accM[d, sl], r)
                                accN[d, sl] = jnp.minimum(accN[d, sl], r)
                                return 0
                            lax.fori_loop(0, NJ, upd, 0, unroll=8)
                            if with_deg:
                                accD[d, pl.ds(0, 16)] += ones16
                            return 0
                        lax.fori_loop(0, 16, pe, 0)
                    return 0

                lax.fori_loop(0, ngroups, do_group, 0)

            issue_idx(0, bufs[0])
            fix_and_gather(0, bufs[0])
            issue_idx(1, bufs[1])

            def pipe(k2, _):
                for ph in range(2):
                    k = k2 * 2 + ph
                    B = bufs[ph]
                    Bn = bufs[1 - ph]

                    @pl.when(k < nchunks)
                    def _():
                        pltpu.make_async_copy(
                            x_hbm.at[B[0]], B[2].at[pl.ds(0, CHUNK)],
                            B[4]).wait()
                        fix_and_gather(k + 1, Bn)
                        compute_chunk(k, B)
                        issue_idx(k + 2, B)
                return 0

            lax.fori_loop(0, lax.div(nchunks + 1, 2), pipe, 0)

            nb = b * NPP
            pltpu.sync_copy(accS.at[pl.ds(0, NPP)], oS.at[pl.ds(nb, NPP)])
            pltpu.sync_copy(accQ.at[pl.ds(0, NPP)], oQ.at[pl.ds(nb, NPP)])
            pltpu.sync_copy(accM.at[pl.ds(0, NPP)], oM.at[pl.ds(nb, NPP)])
            pltpu.sync_copy(accN.at[pl.ds(0, NPP)], oN.at[pl.ds(nb, NPP)])
            if with_deg:
                pltpu.sync_copy(accD.at[pl.ds(0, NPP)], oD.at[pl.ds(nb, NPP)])
            return 0

        lax.fori_loop(0, 4, do_bucket, 0)

    return agg(x, src_s, dstl_s, off)


def _tc_scalars(deg):
    def body(deg_ref, ic_ref, s1_ref, s2_ref, hb_ref):
        dg = deg_ref[...]
        node = lax.broadcasted_iota(jnp.int32, (1, NTC), 1)
        real = node < N_NODES
        lg = jnp.where(real, jnp.log(jnp.where(real, dg, 0.0) + 1.0), 0.0)
        mean_lg = jnp.sum(lg) / N_NODES
        scale = lg / jnp.clip(mean_lg, 1e-6, None)
        s2 = 1.0 / jnp.clip(scale, 1e-2, None)
        ic = 1.0 / jnp.clip(dg, 1.0, None)
        hb = jnp.where(dg > 0, 1.0, 0.0)
        ic_ref[...] = ic.reshape(NTC, 1)
        s1_ref[...] = scale.reshape(NTC, 1)
        s2_ref[...] = s2.reshape(NTC, 1)
        hb_ref[...] = hb.reshape(NTC, 1)

    o = jax.ShapeDtypeStruct((NTC, 1), jnp.float32)
    return pl.pallas_call(body, out_shape=(o, o, o, o))(deg)


def _tc_dense(x, S, Q, MX, MN, ic, s1, s2, hb, Wself, Wf0, Wf1, Wf2, b):
    BLK = 512

    def body(x_ref, S_ref, Q_ref, MX_ref, MN_ref, ic_ref, s1_ref, s2_ref,
             hb_ref, Wself_ref, Wf0_ref, Wf1_ref, Wf2_ref, b_ref, o_ref):
        xv = x_ref[...]
        ic_ = ic_ref[...]
        hb_ = hb_ref[...]
        mean = S_ref[...] * ic_
        sqm = Q_ref[...] * ic_
        std = jnp.sqrt(jnp.clip(sqm - mean * mean, 0.0, None))
        mx = jnp.where(hb_ > 0, MX_ref[...], 0.0)
        mn = jnp.where(hb_ > 0, MN_ref[...], 0.0)
        feat = jnp.concatenate([mean, mx, mn, std], axis=1)
        acc = jnp.dot(xv, Wself_ref[...], preferred_element_type=jnp.float32)
        acc += jnp.dot(feat, Wf0_ref[...], preferred_element_type=jnp.float32)
        acc += jnp.dot(feat * s1_ref[...], Wf1_ref[...],
                       preferred_element_type=jnp.float32)
        acc += jnp.dot(feat * s2_ref[...], Wf2_ref[...],
                       preferred_element_type=jnp.float32)
        o_ref[...] = jax.nn.relu(acc + b_ref[...]) + xv

    n_blk = NTC // BLK
    row = lambda i: (i, 0)
    fix = lambda i: (0, 0)
    rspec = pl.BlockSpec((BLK, D), row)
    sspec = pl.BlockSpec((BLK, 1), row)
    return pl.pallas_call(
        body,
        grid=(n_blk,),
        in_specs=[rspec, rspec, rspec, rspec, rspec, sspec, sspec, sspec,
                  sspec,
                  pl.BlockSpec((D, D), fix),
                  pl.BlockSpec((4 * D, D), fix),
                  pl.BlockSpec((4 * D, D), fix),
                  pl.BlockSpec((4 * D, D), fix),
                  pl.BlockSpec((1, D), fix)],
        out_specs=rspec,
        out_shape=jax.ShapeDtypeStruct((NTC, D), jnp.float32),
    )(x, S, Q, MX, MN, ic, s1, s2, hb, Wself, Wf0, Wf1, Wf2, b)


def kernel(input_embeds, edge_index, input_index, W0, b0, W1, b1, W2, b2):
    src = jnp.concatenate([edge_index[0], edge_index[1]])
    dst = jnp.concatenate([edge_index[1], edge_index[0]])

    perm = jnp.argsort(dst)
    src_s = src[perm]
    dst_s = dst[perm]
    dstl_s = dst_s % NPP
    bounds = jnp.arange(0, NB + 1, dtype=jnp.int32) * NPP
    off = jnp.searchsorted(dst_s, bounds, side="left").astype(jnp.int32)
    off = jnp.concatenate([off, jnp.full((160 - NB - 1,), E2, jnp.int32)])

    ii = jnp.arange(NUM_INPUT, dtype=jnp.int32)
    winner = jax.ops.segment_max(ii, input_index, num_segments=N_NODES)
    winner = jnp.where(winner >= 0, winner, NUM_INPUT)
    emb_pad = jnp.concatenate(
        [input_embeds, jnp.zeros((1, D), jnp.float32)], axis=0)
    x0 = emb_pad[winner]
    x0 = jnp.concatenate([x0, jnp.zeros((NTC - N_NODES, D), jnp.float32)])

    layers = []
    for W, b in ((W0, b0), (W1, b1), (W2, b2)):
        Wself = W[:D]
        Wf = W[D:].reshape(4 * D, 3, D)
        layers.append((Wself, Wf[:, 0, :], Wf[:, 1, :], Wf[:, 2, :],
                       b.reshape(1, D)))

    x = x0
    ic = s1 = s2 = hb = None
    for li, (Wself, Wf0, Wf1, Wf2, bb) in enumerate(layers):
        if li == 0:
            S, Q, MX, MN, Dg = _sc_aggregate(x, src_s, dstl_s, off, True)
            deg_p = Dg[:, 0].reshape(1, NTC)
            ic, s1, s2, hb = _tc_scalars(deg_p)
        else:
            S, Q, MX, MN = _sc_aggregate(x, src_s, dstl_s, off, False)
        x = _tc_dense(x, S, Q, MX, MN, ic, s1, s2, hb,
                      Wself, Wf0, Wf1, Wf2, bb)

    return x[:N_NODES]
